# baseline (speedup 1.0000x reference)
import jax
import jax.numpy as jnp
from jax import lax
from jax.experimental import pallas as pl
from jax.experimental.pallas import tpu as pltpu
from jax.experimental.pallas import tpu_sc as plsc

_BATCH = 16384
_K = 16

_info = plsc.get_sparse_core_info()
_NC, _NS, _L = _info.num_cores, _info.num_subcores, _info.num_lanes
_NW = _NC * _NS
_BPW = _BATCH // _NW


def _ncf_body(u_hbm, v_hbm, w_hbm, h_hbm, lin_hbm, out_hbm,
              uidx_v, vidx_v, urows_v, vrows_v, lin_v, out_v, sem):
    wid = lax.axis_index("s") * _NC + lax.axis_index("c")
    base = wid * _BPW

    pltpu.sync_copy(lin_hbm, lin_v)
    pltpu.sync_copy(u_hbm.at[pl.ds(base, _BPW)], uidx_v)
    pltpu.sync_copy(v_hbm.at[pl.ds(base, _BPW)], vidx_v)

    cu = pltpu.async_copy(w_hbm.at[uidx_v], urows_v, sem)
    cv = pltpu.async_copy(h_hbm.at[vidx_v], vrows_v, sem)
    cu.wait()
    cv.wait()

    wu_vec = lin_v[pl.ds(0, _L)]
    wv_vec = lin_v[pl.ds(_K, _L)]
    wb_vec = lin_v[pl.ds(2 * _K, _L)]
    wk = [wu_vec[k] for k in range(_K)] + [wv_vec[k] for k in range(_K)]
    lb = wb_vec[0]

    def blk_body(blk, carry):
        rbase = blk * _L
        rows = rbase + lax.iota(jnp.int32, _L)
        acc = jnp.full((_L,), 0.0, jnp.float32)
        for k in range(_K):
            colk = jnp.full((_L,), k, jnp.int32)
            uc = plsc.load_gather(urows_v, [rows, colk])
            vc = plsc.load_gather(vrows_v, [rows, colk])
            acc = acc + uc * wk[k] + vc * wk[_K + k]
        z = acc + lb
        out_v[pl.ds(rbase, _L)] = 1.0 / (1.0 + jnp.exp(-z))
        return carry

    lax.fori_loop(0, _BPW // _L, blk_body, 0)

    pltpu.sync_copy(out_v, out_hbm.at[pl.ds(base, _BPW)])


_ncf_sc = pl.kernel(
    _ncf_body,
    mesh=plsc.VectorSubcoreMesh(core_axis_name="c", subcore_axis_name="s"),
    out_type=jax.ShapeDtypeStruct((_BATCH,), jnp.float32),
    scratch_types=[
        pltpu.VMEM((_BPW,), jnp.int32),
        pltpu.VMEM((_BPW,), jnp.int32),
        pltpu.VMEM((_BPW, _K), jnp.float32),
        pltpu.VMEM((_BPW, _K), jnp.float32),
        pltpu.VMEM((48,), jnp.float32),
        pltpu.VMEM((_BPW,), jnp.float32),
        pltpu.SemaphoreType.DMA,
    ],
    compiler_params=pltpu.CompilerParams(
        needs_layout_passes=False, use_tc_tiling_on_sc=False),
)


@jax.jit
def kernel(x, W, H, lin_w, lin_b):
    u_idx = x[:, 0]
    v_idx = x[:, 1]
    lin_all = jnp.concatenate(
        [lin_w.reshape(-1), lin_b.reshape(-1), jnp.zeros((15,), jnp.float32)])
    return _ncf_sc(u_idx, v_idx, W, H, lin_all)
